# Initial kernel scaffold; baseline (speedup 1.0000x reference)
import jax
import jax.numpy as jnp
from jax.experimental import pallas as pl


def kernel(inputs, table):
    raise NotImplementedError("write your pallas kernel here")


# speedup vs baseline: 5.2456x; 5.2456x over previous
import functools

import jax
import jax.numpy as jnp
from jax import lax
from jax.experimental import pallas as pl
from jax.experimental.pallas import tpu as pltpu
from jax.experimental.pallas import tpu_sc as plsc

NUM_NUCLEOTIDES = 16
PAD_IDX = NUM_NUCLEOTIDES - 1

_NC = 2
_NS = 16
_NW = _NC * _NS

_GATHER_W = 128
_GATHERS_PER_STEP = 8
_CHUNK = _GATHER_W * _GATHERS_PER_STEP


@functools.lru_cache(maxsize=None)
def _make_sc_gather(total: int, dim: int):
    assert total % (_NW * _CHUNK) == 0
    per_w = total // _NW
    steps = per_w // _CHUNK

    mesh = plsc.VectorSubcoreMesh(core_axis_name="c", subcore_axis_name="s")

    @functools.partial(
        pl.kernel,
        mesh=mesh,
        out_type=jax.ShapeDtypeStruct((total, dim), jnp.float32),
        scratch_types=[
            pltpu.VMEM((_CHUNK,), jnp.int32),
            pltpu.VMEM((_CHUNK, dim), jnp.float32),
            pltpu.VMEM_SHARED((NUM_NUCLEOTIDES, dim), jnp.float32),
            pltpu.SemaphoreType.DMA,
        ],
        compiler_params=pltpu.CompilerParams(use_tc_tiling_on_sc=False),
    )
    def sc_kernel(idx_hbm, table_hbm, out_hbm, idx_v, rows_v, table_sh, sem):
        sid = lax.axis_index("s")
        wid = sid * _NC + lax.axis_index("c")
        base = wid * per_w

        @pl.when(sid == 0)
        def _():
            pltpu.sync_copy(table_hbm, table_sh)
        plsc.subcore_barrier()

        def step(t, carry):
            pltpu.sync_copy(
                idx_hbm.at[pl.ds(base + t * _CHUNK, _CHUNK)],
                idx_v)
            handles = [
                pltpu.async_copy(
                    table_sh.at[idx_v.at[pl.ds(j * _GATHER_W, _GATHER_W)]],
                    rows_v.at[pl.ds(j * _GATHER_W, _GATHER_W)],
                    sem)
                for j in range(_GATHERS_PER_STEP)
            ]
            for h in handles:
                h.wait()
            pltpu.sync_copy(
                rows_v, out_hbm.at[pl.ds(base + t * _CHUNK, _CHUNK)])
            return carry

        lax.fori_loop(0, steps, step, 0)

    return sc_kernel


def kernel(inputs, table):
    n_rows, n_cols = inputs.shape
    dim = table.shape[1]
    total = n_rows * n_cols
    table_z = table.at[PAD_IDX].set(0.0)
    idx_flat = inputs.astype(jnp.int32).reshape(total)
    out = _make_sc_gather(total, dim)(idx_flat, table_z)
    return out.reshape(n_rows, n_cols, dim)


# speedup vs baseline: 15.0128x; 2.8620x over previous
import functools

import jax
import jax.numpy as jnp
from jax import lax
from jax.experimental import pallas as pl
from jax.experimental.pallas import tpu as pltpu
from jax.experimental.pallas import tpu_sc as plsc

NUM_NUCLEOTIDES = 16
PAD_IDX = NUM_NUCLEOTIDES - 1

_NC = 2
_NS = 16
_NW = _NC * _NS
_L = 16

_TDIM = 8
_BDIM = 128


@functools.lru_cache(maxsize=None)
def _make_sc_gather(n_cols: int, n_rows: int, dim: int):
    n_dt = dim // _TDIM
    n_bb = n_rows // _BDIM
    bb_per_w = n_bb // _NW
    bpw = bb_per_w * _BDIM
    assert bb_per_w * _NW == n_bb and n_dt * _TDIM == dim

    mesh = plsc.VectorSubcoreMesh(core_axis_name="c", subcore_axis_name="s")

    @functools.partial(
        pl.kernel,
        mesh=mesh,
        out_type=jax.ShapeDtypeStruct(
            (n_cols, n_dt, _NW, bb_per_w, _TDIM, _BDIM), jnp.float32),
        scratch_types=[
            pltpu.VMEM((dim, NUM_NUCLEOTIDES), jnp.float32),
            pltpu.VMEM((bpw,), jnp.int32),
            pltpu.VMEM((2, bb_per_w, _TDIM, _BDIM), jnp.float32),
            pltpu.SemaphoreType.DMA,
            pltpu.SemaphoreType.DMA,
        ],
    )
    def sc_kernel(idx_hbm, ttab_hbm, out_hbm, ttab_v, idx_v, rows_v,
                  sem0, sem1):
        wid = lax.axis_index("s") * _NC + lax.axis_index("c")
        sems = (sem0, sem1)
        pltpu.sync_copy(ttab_hbm, ttab_v)

        def out_slot(j, dt):
            return out_hbm.at[j, dt, wid]

        def step(j, carry):
            pltpu.sync_copy(idx_hbm.at[j, pl.ds(wid * bpw, bpw)], idx_v)
            for dt in range(n_dt):
                p = dt % 2
                if dt >= 2:
                    pltpu.make_async_copy(
                        rows_v.at[p], out_slot(j, dt - 2), sems[p]).wait()
                tc = [ttab_v[dt * _TDIM + d, :] for d in range(_TDIM)]
                for bb in range(bb_per_w):
                    for g in range(_BDIM // _L):
                        iv = idx_v[pl.ds(bb * _BDIM + g * _L, _L)]
                        for d in range(_TDIM):
                            rows_v[p, bb, d, pl.ds(g * _L, _L)] = (
                                jnp.take(tc[d], iv, axis=0))
                pltpu.async_copy(rows_v.at[p], out_slot(j, dt), sems[p])
            pltpu.make_async_copy(
                rows_v.at[0], out_slot(j, n_dt - 2), sems[0]).wait()
            pltpu.make_async_copy(
                rows_v.at[1], out_slot(j, n_dt - 1), sems[1]).wait()
            return carry

        lax.fori_loop(0, n_cols, step, 0)

    return sc_kernel


def kernel(inputs, table):
    n_rows, n_cols = inputs.shape
    dim = table.shape[1]
    table_t = table.at[PAD_IDX].set(0.0).T
    idx_t = inputs.astype(jnp.int32).T
    out6 = _make_sc_gather(n_cols, n_rows, dim)(idx_t, table_t)
    out = out6.reshape(n_cols, dim // _TDIM, n_rows // _BDIM, _TDIM, _BDIM)
    out = out.transpose(2, 4, 0, 1, 3).reshape(n_rows, n_cols, dim)
    return out
